# baseline (speedup 1.0000x reference)
import jax
import jax.numpy as jnp
from jax import lax
from jax.experimental import pallas as pl
from jax.experimental.pallas import tpu as pltpu
from jax.experimental.pallas import tpu_sc as plsc

N = 10000
E = 320000
D = 128
DE = 16
H = 8
DH = 16
EPS = 1e-3

NW = 32
CH = 128
CPW = 80
E_PAD = NW * CH * CPW
TW = 144
TWB = 160
NPT = N // 16

EB = 1600
NB = 1000

_f32 = jnp.float32
_bf16 = jnp.bfloat16


def _k1a_body(na_ref, w1, w2, wcat, batf, ar16, as16, ts_o, tr_o):
    na = na_ref[...]
    ps = lax.dot(na, w1[...], preferred_element_type=_f32)
    pr = lax.dot(na, w2[...], preferred_element_type=_f32)
    wn = lax.dot(na, wcat[...], preferred_element_type=_f32) + batf[...]
    ls16 = lax.dot(wn, as16[...], preferred_element_type=_f32)
    lr16 = lax.dot(wn, ar16[...], preferred_element_type=_f32)
    blk = na.shape[0]
    zer16 = jnp.zeros((blk, 16), _f32)
    ts_o[...] = jnp.concatenate([ps, ls16, zer16], axis=1).astype(_bf16)
    tr_o[...] = jnp.concatenate([pr, lr16, zer16], axis=1).astype(_bf16)


def _k1a_call(na, w1, w2, wcat, batf, ar16, as16):
    nblk = 2000
    grid = (N // nblk,)
    const = lambda shape: pl.BlockSpec(shape, lambda i: (0, 0))
    return pl.pallas_call(
        _k1a_body,
        grid=grid,
        in_specs=[
            pl.BlockSpec((nblk, D), lambda i: (i, 0)),
            const((D, D)), const((D, D)), const((D, D)), const((1, D)),
            const((D, 16)), const((D, 16)),
        ],
        out_specs=[
            pl.BlockSpec((nblk, TWB), lambda i: (i, 0)),
            pl.BlockSpec((nblk, TWB), lambda i: (i, 0)),
        ],
        out_shape=[
            jax.ShapeDtypeStruct((N, TWB), _bf16),
            jax.ShapeDtypeStruct((N, TWB), _bf16),
        ],
    )(na, w1, w2, wcat, batf, ar16, as16)


def _k1b_body(na_ref, wcat, batf, wnod, bnod, wne_o, nur_o, nst_o, acc):
    i = pl.program_id(0)
    na = na_ref[...]
    wn = lax.dot(na, wcat[...], preferred_element_type=_f32) + batf[...]
    blk = na.shape[0]
    ones8 = jnp.ones((blk, 8), _f32)
    zer8 = jnp.zeros((blk, 8), _f32)
    wne_o[...] = jnp.concatenate([wn, ones8, zer8], axis=1)
    nu = jnp.maximum(lax.dot(na, wnod[...], preferred_element_type=_f32)
                     + bnod[...], 0.0)
    nur_o[...] = nu
    s0 = jnp.sum(nu, axis=0, keepdims=True)
    s1 = jnp.sum(nu * nu, axis=0, keepdims=True)
    st = jnp.concatenate([s0, s1], axis=0)

    @pl.when(i == 0)
    def _():
        acc[...] = jnp.zeros_like(acc)

    acc[...] += st

    @pl.when(i == pl.num_programs(0) - 1)
    def _():
        nst_o[...] = acc[...]


def _k1b_call(na, wcat, batf, wnod, bnod):
    nblk = 2000
    grid = (N // nblk,)
    const = lambda shape: pl.BlockSpec(shape, lambda i: (0, 0))
    return pl.pallas_call(
        _k1b_body,
        grid=grid,
        in_specs=[
            pl.BlockSpec((nblk, D), lambda i: (i, 0)),
            const((D, D)), const((1, D)), const((D, D)), const((1, D)),
        ],
        out_specs=[
            pl.BlockSpec((nblk, TW), lambda i: (i, 0)),
            pl.BlockSpec((nblk, D), lambda i: (i, 0)),
            pl.BlockSpec((2, D), lambda i: (0, 0)),
        ],
        out_shape=[
            jax.ShapeDtypeStruct((N, TW), _f32),
            jax.ShapeDtypeStruct((N, D), _f32),
            jax.ShapeDtypeStruct((2, D), _f32),
        ],
        scratch_shapes=[pltpu.VMEM((2, D), _f32)],
    )(na, wcat, batf, wnod, bnod)


KB2 = 8


def _k2_adds(bs, br):
    @pl.loop(0, CH)
    def _(e):
        for k in range(5):
            sl = pl.ds(k * 32, 32)
            bs[e, sl] = bs[e, sl] + br[e, sl]


def _k2_adds(bs, br):
    @pl.loop(0, CH)
    def _(e):
        for k in range(5):
            sl = pl.ds(k * 32, 32)
            bs[e, sl] = bs[e, sl] + br[e, sl]


def _k2_body(ts_hbm, tr_hbm, is2d, ir2d, gcat_hbm,
             isb, irb, bs, br, ts_sp, tr_sp, sg1, sg2):
    cid = lax.axis_index("c")
    sid = lax.axis_index("s")
    wid = sid * 2 + cid

    tb = sid * NPT
    pltpu.sync_copy(ts_hbm.at[pl.ds(tb, NPT)], ts_sp.at[pl.ds(tb, NPT)])
    pltpu.sync_copy(tr_hbm.at[pl.ds(tb, NPT)], tr_sp.at[pl.ds(tb, NPT)])
    plsc.subcore_barrier()

    @pl.loop(0, CPW // KB2)
    def _(ob):
        crow = wid * CPW + ob * KB2
        pltpu.sync_copy(is2d.at[pl.ds(crow, KB2)], isb)
        pltpu.sync_copy(ir2d.at[pl.ds(crow, KB2)], irb)
        for j in range(KB2):
            base = (crow + j) * CH
            c1 = pltpu.async_copy(ts_sp.at[isb.at[j]], bs, sg1)
            c2 = pltpu.async_copy(tr_sp.at[irb.at[j]], br, sg2)
            c1.wait()
            c2.wait()
            _k2_adds(bs, br)
            pltpu.sync_copy(bs, gcat_hbm.at[pl.ds(base, CH)])


def _k2_call(ts, tr, is2d, ir2d):
    mesh = plsc.VectorSubcoreMesh(core_axis_name="c", subcore_axis_name="s")
    f = pl.kernel(
        _k2_body,
        out_type=jax.ShapeDtypeStruct((E_PAD, TWB), _bf16),
        mesh=mesh,
        compiler_params=pltpu.CompilerParams(use_tc_tiling_on_sc=False),
        scratch_types=[
            pltpu.VMEM((KB2, CH), jnp.int32),
            pltpu.VMEM((KB2, CH), jnp.int32),
            pltpu.VMEM((CH, TWB), _bf16),
            pltpu.VMEM((CH, TWB), _bf16),
            pltpu.VMEM_SHARED((N, TWB), _bf16),
            pltpu.VMEM_SHARED((N, TWB), _bf16),
            pltpu.SemaphoreType.DMA,
            pltpu.SemaphoreType.DMA,
        ],
    )
    return f(ts, tr, is2d, ir2d)


def _k3_body(g_ref, ea_ref, w3, bedge, o_ref, acc):
    i = pl.program_id(0)
    g = g_ref[...].astype(_f32)
    q = lax.dot(ea_ref[...], w3[...], preferred_element_type=_f32)
    x = jnp.maximum(g + q + bedge[...], 0.0)
    s0 = jnp.sum(x, axis=0, keepdims=True)
    s1 = jnp.sum(x * x, axis=0, keepdims=True)
    st = jnp.concatenate([s0, s1], axis=0)

    @pl.when(i == 0)
    def _():
        acc[...] = jnp.zeros_like(acc)

    acc[...] += st

    @pl.when(i == pl.num_programs(0) - 1)
    def _():
        o_ref[...] = acc[...]


def _k3_call(gcat, ea, w3, bedge):
    grid = (E // EB,)
    const = lambda shape: pl.BlockSpec(shape, lambda i: (0, 0))
    return pl.pallas_call(
        _k3_body,
        grid=grid,
        in_specs=[
            pl.BlockSpec((EB, D), lambda i: (i, 0)),
            pl.BlockSpec((EB, DE), lambda i: (i, 0)),
            const((DE, D)), const((1, D)),
        ],
        out_specs=pl.BlockSpec((2, D), lambda i: (0, 0)),
        out_shape=jax.ShapeDtypeStruct((2, D), _f32),
        scratch_shapes=[pltpu.VMEM((2, D), _f32)],
    )(gcat, ea, w3, bedge)


def _k4a_body(g_ref, ea_ref, st_ref, w3, bedge, gam, bet, aet16, mask16,
              ex_o):
    s = st_ref[...]
    mu = s[0:1, :] * (1.0 / E)
    ms = s[1:2, :] * (1.0 / E)
    var = ms - mu * mu
    c = gam[...] * lax.rsqrt(var + EPS)
    d = bet[...] - c * mu
    gfull = g_ref[...]
    g = gfull[:, :D].astype(_f32)
    lsum16 = gfull[:, D:D + 16].astype(_f32)
    q = lax.dot(ea_ref[...], w3[...], preferred_element_type=_f32)
    x = jnp.maximum(g + q + bedge[...], 0.0)
    y = c * x + d
    u16 = lax.dot(y, aet16[...], preferred_element_type=_f32)
    v16 = lsum16 + u16
    lg16 = jnp.where(v16 >= 0.0, v16, 0.2 * v16)
    ex_o[...] = jnp.exp(lg16) * mask16[...]


def _k4a_call(gcat, ea, estats, w3, bedge, gam, bet, aet16, mask16):
    grid = (E // EB,)
    const = lambda shape: pl.BlockSpec(shape, lambda i: (0, 0))
    return pl.pallas_call(
        _k4a_body,
        grid=grid,
        in_specs=[
            pl.BlockSpec((EB, TWB), lambda i: (i, 0)),
            pl.BlockSpec((EB, DE), lambda i: (i, 0)),
            const((2, D)), const((DE, D)), const((1, D)),
            const((1, D)), const((1, D)), const((D, 16)), const((1, 16)),
        ],
        out_specs=pl.BlockSpec((EB, 16), lambda i: (i, 0)),
        out_shape=jax.ShapeDtypeStruct((E, 16), _f32),
    )(gcat, ea, estats, w3, bedge, gam, bet, aet16, mask16)


def _k4b_body(g_ref, ea_ref, st_ref, w3, bedge, gam, bet, y_o):
    s = st_ref[...]
    mu = s[0:1, :] * (1.0 / E)
    ms = s[1:2, :] * (1.0 / E)
    var = ms - mu * mu
    c = gam[...] * lax.rsqrt(var + EPS)
    d = bet[...] - c * mu
    g = g_ref[...].astype(_f32)
    q = lax.dot(ea_ref[...], w3[...], preferred_element_type=_f32)
    x = jnp.maximum(g + q + bedge[...], 0.0)
    y_o[...] = c * x + d


def _k4b_call(gcat, ea, estats, w3, bedge, gam, bet):
    grid = (E // EB,)
    const = lambda shape: pl.BlockSpec(shape, lambda i: (0, 0))
    return pl.pallas_call(
        _k4b_body,
        grid=grid,
        in_specs=[
            pl.BlockSpec((EB, D), lambda i: (i, 0)),
            pl.BlockSpec((EB, DE), lambda i: (i, 0)),
            const((2, D)), const((DE, D)), const((1, D)),
            const((1, D)), const((1, D)),
        ],
        out_specs=pl.BlockSpec((EB, D), lambda i: (i, 0)),
        out_shape=jax.ShapeDtypeStruct((E, D), _f32),
    )(gcat, ea, estats, w3, bedge, gam, bet)


def _lane_bcast(vec, lane):
    dn = lax.GatherDimensionNumbers(
        offset_dims=(), collapsed_slice_dims=(0,), start_index_map=(0,))
    idx = jnp.full((16, 1), lane, jnp.int32)
    return lax.gather(vec, idx, dn, slice_sizes=(1,),
                      mode=lax.GatherScatterMode.PROMISE_IN_BOUNDS)


KB5 = 4
CPA = 100
CPB = 60


def _k5_body(wne_hbm, ex2d, is2d, ir2d, pool_out,
             isb, irb, wbuf, exb, psp, sem1, ssem):
    cid = lax.axis_index("c")
    sid = lax.axis_index("s")
    wid = sid * 2 + cid
    z16 = jnp.zeros((16,), _f32)

    @pl.loop(0, CH)
    def _(e):
        for k in range(9):
            wbuf[e, pl.ds(k * 16, 16)] = z16

    tb = sid * NPT
    for j in range(5):
        pltpu.sync_copy(wbuf.at[pl.ds(0, 125)],
                        psp.at[pl.ds(tb + j * 125, 125)])
    plsc.subcore_barrier()

    start_chunk = jnp.where(cid == 0, sid * CPA, 16 * CPA + sid * CPB)
    nbat = jnp.where(cid == 0, CPA // KB5, CPB // KB5)

    @pl.loop(0, nbat)
    def _(ob):
        crow = start_chunk + ob * KB5

        @pl.when(ob > 0)
        def _():
            pltpu.make_async_copy(wbuf, psp.at[irb.at[0]], ssem).wait()

        pltpu.sync_copy(is2d.at[pl.ds(crow, KB5)], isb)
        pltpu.sync_copy(ir2d.at[pl.ds(crow, KB5)], irb)
        pltpu.sync_copy(ex2d.at[pl.ds(crow, KB5)], exb)
        for j in range(KB5):
            if j > 0:
                pltpu.make_async_copy(wbuf, psp.at[irb.at[j]], ssem).wait()
            pltpu.async_copy(wne_hbm.at[isb.at[j]], wbuf, sem1).wait()

            @pl.loop(0, CH)
            def _(e):
                vec = exb[j, pl.ds(e * 16, 16)]
                for h in range(8):
                    bc = _lane_bcast(vec, h)
                    sl = pl.ds(h * 16, 16)
                    wbuf[e, sl] = wbuf[e, sl] * bc
                sl = pl.ds(D, 16)
                wbuf[e, sl] = wbuf[e, sl] * vec

            pltpu.async_copy(wbuf, psp.at[irb.at[j]], ssem, add=True)

    pltpu.make_async_copy(wbuf, psp.at[irb.at[0]], ssem).wait()
    plsc.subcore_barrier()
    for j in range(5):
        pltpu.sync_copy(psp.at[pl.ds(tb + j * 125, 125)],
                        pool_out.at[cid, pl.ds(tb + j * 125, 125)])


def _k5_call(wne, ex2d, is2d, ir2d):
    mesh = plsc.VectorSubcoreMesh(core_axis_name="c", subcore_axis_name="s")
    f = pl.kernel(
        _k5_body,
        out_type=jax.ShapeDtypeStruct((2, N, TW), _f32),
        mesh=mesh,
        compiler_params=pltpu.CompilerParams(use_tc_tiling_on_sc=False),
        scratch_types=[
            pltpu.VMEM((KB5, CH), jnp.int32),
            pltpu.VMEM((KB5, CH), jnp.int32),
            pltpu.VMEM((CH, TW), _f32),
            pltpu.VMEM((KB5, CH * 16), _f32),
            pltpu.VMEM_SHARED((N, TW), _f32),
            pltpu.SemaphoreType.DMA,
            pltpu.SemaphoreType.DMA,
        ],
    )
    return f(wne, ex2d, is2d, ir2d)


def _k6_body(pp_ref, nur_ref, nst_ref, gn, bn2, oh16, gruw, gruu, grub,
             o_ref):
    p = pp_ref[0] + pp_ref[1]
    praw = p[:, :D]
    dn16 = p[:, D:TW]
    dexp = lax.dot(dn16, oh16[...], preferred_element_type=_f32) + 1e-9
    att = jnp.maximum(praw / dexp, 0.0)
    s = nst_ref[...]
    mun = s[0:1, :] * (1.0 / N)
    msn = s[1:2, :] * (1.0 / N)
    varn = msn - mun * mun
    cn = gn[...] * lax.rsqrt(varn + EPS)
    dnn = bn2[...] - cn * mun
    nu = cn * nur_ref[...] + dnn
    gx = lax.dot(att, gruw[...], preferred_element_type=_f32) + grub[...]
    gh = lax.dot(nu, gruu[...], preferred_element_type=_f32)
    z = jax.nn.sigmoid(gx[:, :D] + gh[:, :D])
    r = jax.nn.sigmoid(gx[:, D:2 * D] + gh[:, D:2 * D])
    ht = jnp.tanh(gx[:, 2 * D:] + r * gh[:, 2 * D:])
    o_ref[...] = z * nu + (1.0 - z) * ht


def _k6_call(pool_part, nur, nstats, gn, bn2, oh16, gruw, gruu, grub):
    grid = (N // NB,)
    const = lambda shape: pl.BlockSpec(shape, lambda i: tuple(0 for _ in shape))
    return pl.pallas_call(
        _k6_body,
        grid=grid,
        in_specs=[
            pl.BlockSpec((2, NB, TW), lambda i: (0, i, 0)),
            pl.BlockSpec((NB, D), lambda i: (i, 0)),
            const((2, D)), const((1, D)), const((1, D)),
            const((16, D)), const((D, 3 * D)), const((D, 3 * D)),
            const((1, 3 * D)),
        ],
        out_specs=pl.BlockSpec((NB, D), lambda i: (i, 0)),
        out_shape=jax.ShapeDtypeStruct((N, D), _f32),
    )(pool_part, nur, nstats, gn, bn2, oh16, gruw, gruu, grub)


def kernel(node_attributes, edge_attributes, edge_indices, W_edge, b_edge,
           gamma_edge, beta_edge, W_att, b_att, a_att, W_node, b_node,
           gamma_node, beta_node, gru_W, gru_U, gru_b):
    idx_recv = edge_indices[:, 0]
    idx_send = edge_indices[:, 1]
    pad = E_PAD - E
    zpad = jnp.zeros((pad,), jnp.int32)
    is_p = jnp.concatenate([idx_send, zpad])
    ir_p = jnp.concatenate([idx_recv, zpad])

    w1 = W_edge[:D]
    w2 = W_edge[D:2 * D]
    w3 = W_edge[2 * D:]
    wcat = W_att.transpose(1, 0, 2).reshape(D, D)
    batf = b_att.reshape(1, D)

    blkid = jnp.arange(D) // DH
    col = jnp.arange(16)
    onehot = (blkid[:, None] == col[None, :]).astype(_f32)
    v_r = a_att[:, :DH].reshape(-1)
    v_s = a_att[:, DH:2 * DH].reshape(-1)
    ar16 = v_r[:, None] * onehot
    as16 = v_s[:, None] * onehot
    oh16 = onehot.T
    aet16 = jnp.concatenate(
        [a_att[:, 2 * DH:].T, jnp.zeros((D, 8), _f32)], axis=1)
    mask16 = (col < H).astype(_f32).reshape(1, 16)

    bedge = b_edge.reshape(1, D)
    gam = gamma_edge.reshape(1, D)
    bet = beta_edge.reshape(1, D)
    gn = gamma_node.reshape(1, D)
    bn2 = beta_node.reshape(1, D)
    bnod = b_node.reshape(1, D)
    grub = gru_b.reshape(1, 3 * D)

    ts, tr = _k1a_call(node_attributes, w1, w2, wcat, batf, ar16, as16)
    is2d = is_p.reshape(E_PAD // CH, CH)
    ir2d = ir_p.reshape(E_PAD // CH, CH)
    gcat = _k2_call(ts, tr, is2d, ir2d)
    wne, nur, nstats = _k1b_call(node_attributes, wcat, batf, W_node, bnod)
    estats = _k3_call(gcat, edge_attributes, w3, bedge)
    expad = _k4a_call(gcat, edge_attributes, estats, w3, bedge, gam, bet,
                      aet16, mask16)
    y = _k4b_call(gcat, edge_attributes, estats, w3, bedge, gam, bet)
    ex2d = jnp.concatenate(
        [expad, jnp.zeros((pad, 16), _f32)],
        axis=0).reshape(E_PAD // CH, CH * 16)
    pool_part = _k5_call(wne, ex2d, is2d, ir2d)
    node_final = _k6_call(pool_part, nur, nstats, gn, bn2, oh16,
                          gru_W, gru_U, grub)
    return node_final, y


# speedup vs baseline: 1.0721x; 1.0721x over previous
import jax
import jax.numpy as jnp
from jax import lax
from jax.experimental import pallas as pl
from jax.experimental.pallas import tpu as pltpu
from jax.experimental.pallas import tpu_sc as plsc

N = 10000
E = 320000
D = 128
DE = 16
H = 8
DH = 16
EPS = 1e-3

NW = 32
CH = 128
CPW = 80
E_PAD = NW * CH * CPW
TW = 144
TWB = 160
NPT = N // 16

EB = 1600
NB = 1000

_f32 = jnp.float32
_bf16 = jnp.bfloat16


def _k1_body(na_ref, w1, w2, wcat, batf, ar16, as16, wnod, bnod,
             ts_o, tr_o, wne_o, nur_o, nst_o, acc):
    i = pl.program_id(0)
    na = na_ref[...]
    ps = lax.dot(na, w1[...], preferred_element_type=_f32)
    pr = lax.dot(na, w2[...], preferred_element_type=_f32)
    wn = lax.dot(na, wcat[...], preferred_element_type=_f32) + batf[...]
    ls16 = lax.dot(wn, as16[...], preferred_element_type=_f32)
    lr16 = lax.dot(wn, ar16[...], preferred_element_type=_f32)
    blk = na.shape[0]
    zer16 = jnp.zeros((blk, 16), _f32)
    ts_o[...] = jnp.concatenate([ps, ls16, zer16], axis=1).astype(_bf16)
    tr_o[...] = jnp.concatenate([pr, lr16, zer16], axis=1).astype(_bf16)
    ones8 = jnp.ones((blk, 8), _f32)
    zer8 = jnp.zeros((blk, 8), _f32)
    wne_o[...] = jnp.concatenate([wn, ones8, zer8], axis=1)
    nu = jnp.maximum(lax.dot(na, wnod[...], preferred_element_type=_f32)
                     + bnod[...], 0.0)
    nur_o[...] = nu
    s0 = jnp.sum(nu, axis=0, keepdims=True)
    s1 = jnp.sum(nu * nu, axis=0, keepdims=True)
    st = jnp.concatenate([s0, s1], axis=0)

    @pl.when(i == 0)
    def _():
        acc[...] = jnp.zeros_like(acc)

    acc[...] += st

    @pl.when(i == pl.num_programs(0) - 1)
    def _():
        nst_o[...] = acc[...]


def _k1_call(na, w1, w2, wcat, batf, ar16, as16, wnod, bnod):
    nblk = 2000
    grid = (N // nblk,)
    const = lambda shape: pl.BlockSpec(shape, lambda i: (0, 0))
    return pl.pallas_call(
        _k1_body,
        grid=grid,
        in_specs=[
            pl.BlockSpec((nblk, D), lambda i: (i, 0)),
            const((D, D)), const((D, D)), const((D, D)), const((1, D)),
            const((D, 16)), const((D, 16)), const((D, D)), const((1, D)),
        ],
        out_specs=[
            pl.BlockSpec((nblk, TWB), lambda i: (i, 0)),
            pl.BlockSpec((nblk, TWB), lambda i: (i, 0)),
            pl.BlockSpec((nblk, TW), lambda i: (i, 0)),
            pl.BlockSpec((nblk, D), lambda i: (i, 0)),
            pl.BlockSpec((2, D), lambda i: (0, 0)),
        ],
        out_shape=[
            jax.ShapeDtypeStruct((N, TWB), _bf16),
            jax.ShapeDtypeStruct((N, TWB), _bf16),
            jax.ShapeDtypeStruct((N, TW), _f32),
            jax.ShapeDtypeStruct((N, D), _f32),
            jax.ShapeDtypeStruct((2, D), _f32),
        ],
        scratch_shapes=[pltpu.VMEM((2, D), _f32)],
    )(na, w1, w2, wcat, batf, ar16, as16, wnod, bnod)


KB2 = 8


def _k2_adds(bs, br):
    @pl.loop(0, CH)
    def _(e):
        for k in range(5):
            sl = pl.ds(k * 32, 32)
            bs[e, sl] = bs[e, sl] + br[e, sl]


def _k2_body(ts_hbm, tr_hbm, is2d, ir2d, gcat_hbm,
             isb, irb, bs, br, ts_sp, tr_sp, sg1, sg2):
    cid = lax.axis_index("c")
    sid = lax.axis_index("s")
    wid = sid * 2 + cid

    tb = sid * NPT
    pltpu.sync_copy(ts_hbm.at[pl.ds(tb, NPT)], ts_sp.at[pl.ds(tb, NPT)])
    pltpu.sync_copy(tr_hbm.at[pl.ds(tb, NPT)], tr_sp.at[pl.ds(tb, NPT)])
    plsc.subcore_barrier()

    @pl.loop(0, CPW // KB2)
    def _(ob):
        crow = wid * CPW + ob * KB2
        pltpu.sync_copy(is2d.at[pl.ds(crow, KB2)], isb)
        pltpu.sync_copy(ir2d.at[pl.ds(crow, KB2)], irb)
        for j in range(KB2):
            base = (crow + j) * CH
            c1 = pltpu.async_copy(ts_sp.at[isb.at[j]], bs, sg1)
            c2 = pltpu.async_copy(tr_sp.at[irb.at[j]], br, sg2)
            c1.wait()
            c2.wait()
            _k2_adds(bs, br)
            pltpu.sync_copy(bs, gcat_hbm.at[pl.ds(base, CH)])


def _k2_call(ts, tr, is2d, ir2d):
    mesh = plsc.VectorSubcoreMesh(core_axis_name="c", subcore_axis_name="s")
    f = pl.kernel(
        _k2_body,
        out_type=jax.ShapeDtypeStruct((E_PAD, TWB), _bf16),
        mesh=mesh,
        compiler_params=pltpu.CompilerParams(use_tc_tiling_on_sc=False),
        scratch_types=[
            pltpu.VMEM((KB2, CH), jnp.int32),
            pltpu.VMEM((KB2, CH), jnp.int32),
            pltpu.VMEM((CH, TWB), _bf16),
            pltpu.VMEM((CH, TWB), _bf16),
            pltpu.VMEM_SHARED((N, TWB), _bf16),
            pltpu.VMEM_SHARED((N, TWB), _bf16),
            pltpu.SemaphoreType.DMA,
            pltpu.SemaphoreType.DMA,
        ],
    )
    return f(ts, tr, is2d, ir2d)


def _k3_body(g_ref, ea_ref, w3, bedge, o_ref, acc):
    i = pl.program_id(0)
    g = g_ref[...][:, :D].astype(_f32)
    q = lax.dot(ea_ref[...], w3[...], preferred_element_type=_f32)
    x = jnp.maximum(g + q + bedge[...], 0.0)
    s0 = jnp.sum(x, axis=0, keepdims=True)
    s1 = jnp.sum(x * x, axis=0, keepdims=True)
    st = jnp.concatenate([s0, s1], axis=0)

    @pl.when(i == 0)
    def _():
        acc[...] = jnp.zeros_like(acc)

    acc[...] += st

    @pl.when(i == pl.num_programs(0) - 1)
    def _():
        o_ref[...] = acc[...]


def _k3_call(gcat, ea, w3, bedge):
    grid = (E // EB,)
    const = lambda shape: pl.BlockSpec(shape, lambda i: (0, 0))
    return pl.pallas_call(
        _k3_body,
        grid=grid,
        in_specs=[
            pl.BlockSpec((EB, TWB), lambda i: (i, 0)),
            pl.BlockSpec((EB, DE), lambda i: (i, 0)),
            const((DE, D)), const((1, D)),
        ],
        out_specs=pl.BlockSpec((2, D), lambda i: (0, 0)),
        out_shape=jax.ShapeDtypeStruct((2, D), _f32),
        scratch_shapes=[pltpu.VMEM((2, D), _f32)],
    )(gcat, ea, w3, bedge)


def _k4_body(g_ref, ea_ref, st_ref, w3, bedge, gam, bet, aet16, mask16,
             y_o, ex_o):
    s = st_ref[...]
    mu = s[0:1, :] * (1.0 / E)
    ms = s[1:2, :] * (1.0 / E)
    var = ms - mu * mu
    c = gam[...] * lax.rsqrt(var + EPS)
    d = bet[...] - c * mu
    gfull = g_ref[...]
    g = gfull[:, :D].astype(_f32)
    lsum16 = gfull[:, D:D + 16].astype(_f32)
    q = lax.dot(ea_ref[...], w3[...], preferred_element_type=_f32)
    x = jnp.maximum(g + q + bedge[...], 0.0)
    y = c * x + d
    y_o[...] = y
    u16 = lax.dot(y, aet16[...], preferred_element_type=_f32)
    v16 = lsum16 + u16
    lg16 = jnp.where(v16 >= 0.0, v16, 0.2 * v16)
    ex_o[...] = jnp.exp(lg16) * mask16[...]


def _k4_call(gcat, ea, estats, w3, bedge, gam, bet, aet16, mask16):
    grid = (E // EB,)
    const = lambda shape: pl.BlockSpec(shape, lambda i: (0, 0))
    return pl.pallas_call(
        _k4_body,
        grid=grid,
        in_specs=[
            pl.BlockSpec((EB, TWB), lambda i: (i, 0)),
            pl.BlockSpec((EB, DE), lambda i: (i, 0)),
            const((2, D)), const((DE, D)), const((1, D)),
            const((1, D)), const((1, D)), const((D, 16)), const((1, 16)),
        ],
        out_specs=[
            pl.BlockSpec((EB, D), lambda i: (i, 0)),
            pl.BlockSpec((EB, 16), lambda i: (i, 0)),
        ],
        out_shape=[
            jax.ShapeDtypeStruct((E, D), _f32),
            jax.ShapeDtypeStruct((E, 16), _f32),
        ],
    )(gcat, ea, estats, w3, bedge, gam, bet, aet16, mask16)


def _lane_bcast(vec, lane):
    dn = lax.GatherDimensionNumbers(
        offset_dims=(), collapsed_slice_dims=(0,), start_index_map=(0,))
    idx = jnp.full((16, 1), lane, jnp.int32)
    return lax.gather(vec, idx, dn, slice_sizes=(1,),
                      mode=lax.GatherScatterMode.PROMISE_IN_BOUNDS)


KB5 = 4
CPA = 100
CPB = 60


def _k5_body(wne_hbm, ex2d, is2d, ir2d, pool_out,
             isb, irb, wbuf, exb, psp, sem1, ssem):
    cid = lax.axis_index("c")
    sid = lax.axis_index("s")
    wid = sid * 2 + cid
    z16 = jnp.zeros((16,), _f32)

    @pl.loop(0, CH)
    def _(e):
        for k in range(9):
            wbuf[e, pl.ds(k * 16, 16)] = z16

    tb = sid * NPT
    for j in range(5):
        pltpu.sync_copy(wbuf.at[pl.ds(0, 125)],
                        psp.at[pl.ds(tb + j * 125, 125)])
    plsc.subcore_barrier()

    start_chunk = jnp.where(cid == 0, sid * CPA, 16 * CPA + sid * CPB)
    nbat = jnp.where(cid == 0, CPA // KB5, CPB // KB5)

    @pl.loop(0, nbat)
    def _(ob):
        crow = start_chunk + ob * KB5

        @pl.when(ob > 0)
        def _():
            pltpu.make_async_copy(wbuf, psp.at[irb.at[0]], ssem).wait()

        pltpu.sync_copy(is2d.at[pl.ds(crow, KB5)], isb)
        pltpu.sync_copy(ir2d.at[pl.ds(crow, KB5)], irb)
        pltpu.sync_copy(ex2d.at[pl.ds(crow, KB5)], exb)
        for j in range(KB5):
            if j > 0:
                pltpu.make_async_copy(wbuf, psp.at[irb.at[j]], ssem).wait()
            pltpu.async_copy(wne_hbm.at[isb.at[j]], wbuf, sem1).wait()

            @pl.loop(0, CH)
            def _(e):
                vec = exb[j, pl.ds(e * 16, 16)]
                for h in range(8):
                    bc = _lane_bcast(vec, h)
                    sl = pl.ds(h * 16, 16)
                    wbuf[e, sl] = wbuf[e, sl] * bc
                sl = pl.ds(D, 16)
                wbuf[e, sl] = wbuf[e, sl] * vec

            pltpu.async_copy(wbuf, psp.at[irb.at[j]], ssem, add=True)

    pltpu.make_async_copy(wbuf, psp.at[irb.at[0]], ssem).wait()
    plsc.subcore_barrier()
    for j in range(5):
        pltpu.sync_copy(psp.at[pl.ds(tb + j * 125, 125)],
                        pool_out.at[cid, pl.ds(tb + j * 125, 125)])


def _k5_call(wne, ex2d, is2d, ir2d):
    mesh = plsc.VectorSubcoreMesh(core_axis_name="c", subcore_axis_name="s")
    f = pl.kernel(
        _k5_body,
        out_type=jax.ShapeDtypeStruct((2, N, TW), _f32),
        mesh=mesh,
        compiler_params=pltpu.CompilerParams(use_tc_tiling_on_sc=False),
        scratch_types=[
            pltpu.VMEM((KB5, CH), jnp.int32),
            pltpu.VMEM((KB5, CH), jnp.int32),
            pltpu.VMEM((CH, TW), _f32),
            pltpu.VMEM((KB5, CH * 16), _f32),
            pltpu.VMEM_SHARED((N, TW), _f32),
            pltpu.SemaphoreType.DMA,
            pltpu.SemaphoreType.DMA,
        ],
    )
    return f(wne, ex2d, is2d, ir2d)


def _k6_body(pp_ref, nur_ref, nst_ref, gn, bn2, oh16, gruw, gruu, grub,
             o_ref):
    p = pp_ref[0] + pp_ref[1]
    praw = p[:, :D]
    dn16 = p[:, D:TW]
    dexp = lax.dot(dn16, oh16[...], preferred_element_type=_f32) + 1e-9
    att = jnp.maximum(praw / dexp, 0.0)
    s = nst_ref[...]
    mun = s[0:1, :] * (1.0 / N)
    msn = s[1:2, :] * (1.0 / N)
    varn = msn - mun * mun
    cn = gn[...] * lax.rsqrt(varn + EPS)
    dnn = bn2[...] - cn * mun
    nu = cn * nur_ref[...] + dnn
    gx = lax.dot(att, gruw[...], preferred_element_type=_f32) + grub[...]
    gh = lax.dot(nu, gruu[...], preferred_element_type=_f32)
    z = jax.nn.sigmoid(gx[:, :D] + gh[:, :D])
    r = jax.nn.sigmoid(gx[:, D:2 * D] + gh[:, D:2 * D])
    ht = jnp.tanh(gx[:, 2 * D:] + r * gh[:, 2 * D:])
    o_ref[...] = z * nu + (1.0 - z) * ht


def _k6_call(pool_part, nur, nstats, gn, bn2, oh16, gruw, gruu, grub):
    grid = (N // NB,)
    const = lambda shape: pl.BlockSpec(shape, lambda i: tuple(0 for _ in shape))
    return pl.pallas_call(
        _k6_body,
        grid=grid,
        in_specs=[
            pl.BlockSpec((2, NB, TW), lambda i: (0, i, 0)),
            pl.BlockSpec((NB, D), lambda i: (i, 0)),
            const((2, D)), const((1, D)), const((1, D)),
            const((16, D)), const((D, 3 * D)), const((D, 3 * D)),
            const((1, 3 * D)),
        ],
        out_specs=pl.BlockSpec((NB, D), lambda i: (i, 0)),
        out_shape=jax.ShapeDtypeStruct((N, D), _f32),
    )(pool_part, nur, nstats, gn, bn2, oh16, gruw, gruu, grub)


def kernel(node_attributes, edge_attributes, edge_indices, W_edge, b_edge,
           gamma_edge, beta_edge, W_att, b_att, a_att, W_node, b_node,
           gamma_node, beta_node, gru_W, gru_U, gru_b):
    idx_recv = edge_indices[:, 0]
    idx_send = edge_indices[:, 1]
    pad = E_PAD - E
    zpad = jnp.zeros((pad,), jnp.int32)
    is_p = jnp.concatenate([idx_send, zpad])
    ir_p = jnp.concatenate([idx_recv, zpad])

    w1 = W_edge[:D]
    w2 = W_edge[D:2 * D]
    w3 = W_edge[2 * D:]
    wcat = W_att.transpose(1, 0, 2).reshape(D, D)
    batf = b_att.reshape(1, D)

    blkid = jnp.arange(D) // DH
    col = jnp.arange(16)
    onehot = (blkid[:, None] == col[None, :]).astype(_f32)
    v_r = a_att[:, :DH].reshape(-1)
    v_s = a_att[:, DH:2 * DH].reshape(-1)
    ar16 = v_r[:, None] * onehot
    as16 = v_s[:, None] * onehot
    oh16 = onehot.T
    aet16 = jnp.concatenate(
        [a_att[:, 2 * DH:].T, jnp.zeros((D, 8), _f32)], axis=1)
    mask16 = (col < H).astype(_f32).reshape(1, 16)

    bedge = b_edge.reshape(1, D)
    gam = gamma_edge.reshape(1, D)
    bet = beta_edge.reshape(1, D)
    gn = gamma_node.reshape(1, D)
    bn2 = beta_node.reshape(1, D)
    bnod = b_node.reshape(1, D)
    grub = gru_b.reshape(1, 3 * D)

    ts, tr, wne, nur, nstats = _k1_call(
        node_attributes, w1, w2, wcat, batf, ar16, as16, W_node, bnod)
    is2d = is_p.reshape(E_PAD // CH, CH)
    ir2d = ir_p.reshape(E_PAD // CH, CH)
    gcat = _k2_call(ts, tr, is2d, ir2d)
    estats = _k3_call(gcat, edge_attributes, w3, bedge)
    y, expad = _k4_call(gcat, edge_attributes, estats, w3, bedge, gam, bet,
                        aet16, mask16)
    ex2d = jnp.concatenate(
        [expad, jnp.zeros((pad, 16), _f32)],
        axis=0).reshape(E_PAD // CH, CH * 16)
    pool_part = _k5_call(wne, ex2d, is2d, ir2d)
    node_final = _k6_call(pool_part, nur, nstats, gn, bn2, oh16,
                          gru_W, gru_U, grub)
    return node_final, y


# speedup vs baseline: 1.1377x; 1.0611x over previous
import jax
import jax.numpy as jnp
from jax import lax
from jax.experimental import pallas as pl
from jax.experimental.pallas import tpu as pltpu
from jax.experimental.pallas import tpu_sc as plsc

N = 10000
E = 320000
D = 128
DE = 16
H = 8
DH = 16
EPS = 1e-3

NW = 32
CH = 128
CPW = 80
E_PAD = NW * CH * CPW
TW = 144
TWB = 160
NPT = N // 16

EB = 3200
NB = 1000

_f32 = jnp.float32
_bf16 = jnp.bfloat16


def _k1_body(na_ref, w1, w2, wcat, batf, ar16, as16, wnod, bnod,
             ts_o, tr_o, wne_o, nur_o, nst_o, acc):
    i = pl.program_id(0)
    na = na_ref[...]
    ps = lax.dot(na, w1[...], preferred_element_type=_f32)
    pr = lax.dot(na, w2[...], preferred_element_type=_f32)
    wn = lax.dot(na, wcat[...], preferred_element_type=_f32) + batf[...]
    ls16 = lax.dot(wn, as16[...], preferred_element_type=_f32)
    lr16 = lax.dot(wn, ar16[...], preferred_element_type=_f32)
    blk = na.shape[0]
    zer16 = jnp.zeros((blk, 16), _f32)
    ts_o[...] = jnp.concatenate([ps, ls16, zer16], axis=1).astype(_bf16)
    tr_o[...] = jnp.concatenate([pr, lr16, zer16], axis=1).astype(_bf16)
    ones8 = jnp.ones((blk, 8), _f32)
    zer8 = jnp.zeros((blk, 8), _f32)
    wne_o[...] = jnp.concatenate([wn, ones8, zer8], axis=1)
    nu = jnp.maximum(lax.dot(na, wnod[...], preferred_element_type=_f32)
                     + bnod[...], 0.0)
    nur_o[...] = nu
    s0 = jnp.sum(nu, axis=0, keepdims=True)
    s1 = jnp.sum(nu * nu, axis=0, keepdims=True)
    st = jnp.concatenate([s0, s1], axis=0)

    @pl.when(i == 0)
    def _():
        acc[...] = jnp.zeros_like(acc)

    acc[...] += st

    @pl.when(i == pl.num_programs(0) - 1)
    def _():
        nst_o[...] = acc[...]


def _k1_call(na, w1, w2, wcat, batf, ar16, as16, wnod, bnod):
    nblk = 2000
    grid = (N // nblk,)
    const = lambda shape: pl.BlockSpec(shape, lambda i: (0, 0))
    return pl.pallas_call(
        _k1_body,
        grid=grid,
        in_specs=[
            pl.BlockSpec((nblk, D), lambda i: (i, 0)),
            const((D, D)), const((D, D)), const((D, D)), const((1, D)),
            const((D, 16)), const((D, 16)), const((D, D)), const((1, D)),
        ],
        out_specs=[
            pl.BlockSpec((nblk, TWB), lambda i: (i, 0)),
            pl.BlockSpec((nblk, TWB), lambda i: (i, 0)),
            pl.BlockSpec((nblk, TW), lambda i: (i, 0)),
            pl.BlockSpec((nblk, D), lambda i: (i, 0)),
            pl.BlockSpec((2, D), lambda i: (0, 0)),
        ],
        out_shape=[
            jax.ShapeDtypeStruct((N, TWB), _bf16),
            jax.ShapeDtypeStruct((N, TWB), _bf16),
            jax.ShapeDtypeStruct((N, TW), _f32),
            jax.ShapeDtypeStruct((N, D), _f32),
            jax.ShapeDtypeStruct((2, D), _f32),
        ],
        scratch_shapes=[pltpu.VMEM((2, D), _f32)],
    )(na, w1, w2, wcat, batf, ar16, as16, wnod, bnod)


KB2 = 8


def _k2_adds(bs, br):
    @pl.loop(0, CH)
    def _(e):
        for k in range(5):
            sl = pl.ds(k * 32, 32)
            bs[e, sl] = bs[e, sl] + br[e, sl]


def _k2_body(ts_hbm, tr_hbm, is2d, ir2d, gcat_hbm,
             isb, irb, bs, br, ts_sp, tr_sp, sg1, sg2):
    cid = lax.axis_index("c")
    sid = lax.axis_index("s")
    wid = sid * 2 + cid

    tb = sid * NPT
    pltpu.sync_copy(ts_hbm.at[pl.ds(tb, NPT)], ts_sp.at[pl.ds(tb, NPT)])
    pltpu.sync_copy(tr_hbm.at[pl.ds(tb, NPT)], tr_sp.at[pl.ds(tb, NPT)])
    plsc.subcore_barrier()

    @pl.loop(0, CPW // KB2)
    def _(ob):
        crow = wid * CPW + ob * KB2
        pltpu.sync_copy(is2d.at[pl.ds(crow, KB2)], isb)
        pltpu.sync_copy(ir2d.at[pl.ds(crow, KB2)], irb)
        for j in range(KB2):
            base = (crow + j) * CH
            c1 = pltpu.async_copy(ts_sp.at[isb.at[j]], bs, sg1)
            c2 = pltpu.async_copy(tr_sp.at[irb.at[j]], br, sg2)
            c1.wait()
            c2.wait()
            _k2_adds(bs, br)
            pltpu.sync_copy(bs, gcat_hbm.at[pl.ds(base, CH)])


def _k2_call(ts, tr, is2d, ir2d):
    mesh = plsc.VectorSubcoreMesh(core_axis_name="c", subcore_axis_name="s")
    f = pl.kernel(
        _k2_body,
        out_type=jax.ShapeDtypeStruct((E_PAD, TWB), _bf16),
        mesh=mesh,
        compiler_params=pltpu.CompilerParams(use_tc_tiling_on_sc=False),
        scratch_types=[
            pltpu.VMEM((KB2, CH), jnp.int32),
            pltpu.VMEM((KB2, CH), jnp.int32),
            pltpu.VMEM((CH, TWB), _bf16),
            pltpu.VMEM((CH, TWB), _bf16),
            pltpu.VMEM_SHARED((N, TWB), _bf16),
            pltpu.VMEM_SHARED((N, TWB), _bf16),
            pltpu.SemaphoreType.DMA,
            pltpu.SemaphoreType.DMA,
        ],
    )
    return f(ts, tr, is2d, ir2d)


def _k3_body(g_ref, ea_ref, w3, bedge, o_ref, acc):
    i = pl.program_id(0)
    g = g_ref[...][:, :D].astype(_f32)
    q = lax.dot(ea_ref[...], w3[...], preferred_element_type=_f32)
    x = jnp.maximum(g + q + bedge[...], 0.0)
    s0 = jnp.sum(x, axis=0, keepdims=True)
    s1 = jnp.sum(x * x, axis=0, keepdims=True)
    st = jnp.concatenate([s0, s1], axis=0)

    @pl.when(i == 0)
    def _():
        acc[...] = jnp.zeros_like(acc)

    acc[...] += st

    @pl.when(i == pl.num_programs(0) - 1)
    def _():
        o_ref[...] = acc[...]


def _k3_call(gcat, ea, w3, bedge):
    grid = (E // EB,)
    const = lambda shape: pl.BlockSpec(shape, lambda i: (0, 0))
    return pl.pallas_call(
        _k3_body,
        grid=grid,
        in_specs=[
            pl.BlockSpec((EB, TWB), lambda i: (i, 0)),
            pl.BlockSpec((EB, DE), lambda i: (i, 0)),
            const((DE, D)), const((1, D)),
        ],
        out_specs=pl.BlockSpec((2, D), lambda i: (0, 0)),
        out_shape=jax.ShapeDtypeStruct((2, D), _f32),
        scratch_shapes=[pltpu.VMEM((2, D), _f32)],
    )(gcat, ea, w3, bedge)


def _k4_body(g_ref, ea_ref, st_ref, w3, bedge, gam, bet, aet16, mask16,
             y_o, ex_o):
    s = st_ref[...]
    mu = s[0:1, :] * (1.0 / E)
    ms = s[1:2, :] * (1.0 / E)
    var = ms - mu * mu
    c = gam[...] * lax.rsqrt(var + EPS)
    d = bet[...] - c * mu
    gfull = g_ref[...]
    g = gfull[:, :D].astype(_f32)
    lsum16 = gfull[:, D:D + 16].astype(_f32)
    q = lax.dot(ea_ref[...], w3[...], preferred_element_type=_f32)
    x = jnp.maximum(g + q + bedge[...], 0.0)
    y = c * x + d
    y_o[...] = y
    u16 = lax.dot(y, aet16[...], preferred_element_type=_f32)
    v16 = lsum16 + u16
    lg16 = jnp.where(v16 >= 0.0, v16, 0.2 * v16)
    ex_o[...] = jnp.exp(lg16) * mask16[...]


def _k4_call(gcat, ea, estats, w3, bedge, gam, bet, aet16, mask16):
    grid = (E // EB,)
    const = lambda shape: pl.BlockSpec(shape, lambda i: (0, 0))
    return pl.pallas_call(
        _k4_body,
        grid=grid,
        in_specs=[
            pl.BlockSpec((EB, TWB), lambda i: (i, 0)),
            pl.BlockSpec((EB, DE), lambda i: (i, 0)),
            const((2, D)), const((DE, D)), const((1, D)),
            const((1, D)), const((1, D)), const((D, 16)), const((1, 16)),
        ],
        out_specs=[
            pl.BlockSpec((EB, D), lambda i: (i, 0)),
            pl.BlockSpec((EB, 16), lambda i: (i, 0)),
        ],
        out_shape=[
            jax.ShapeDtypeStruct((E, D), _f32),
            jax.ShapeDtypeStruct((E, 16), _f32),
        ],
    )(gcat, ea, estats, w3, bedge, gam, bet, aet16, mask16)


def _lane_bcast(vec, lane):
    dn = lax.GatherDimensionNumbers(
        offset_dims=(), collapsed_slice_dims=(0,), start_index_map=(0,))
    idx = jnp.full((16, 1), lane, jnp.int32)
    return lax.gather(vec, idx, dn, slice_sizes=(1,),
                      mode=lax.GatherScatterMode.PROMISE_IN_BOUNDS)


KB5 = 4
CPA = 100
CPB = 60


def _k5_body(wne_hbm, ex2d, is2d, ir2d, pool_out,
             isb, irb, wbuf, exb, psp, sem1, ssem):
    cid = lax.axis_index("c")
    sid = lax.axis_index("s")
    wid = sid * 2 + cid
    z16 = jnp.zeros((16,), _f32)

    @pl.loop(0, CH)
    def _(e):
        for k in range(9):
            wbuf[e, pl.ds(k * 16, 16)] = z16

    tb = sid * NPT
    for j in range(5):
        pltpu.sync_copy(wbuf.at[pl.ds(0, 125)],
                        psp.at[pl.ds(tb + j * 125, 125)])
    plsc.subcore_barrier()

    start_chunk = jnp.where(cid == 0, sid * CPA, 16 * CPA + sid * CPB)
    nbat = jnp.where(cid == 0, CPA // KB5, CPB // KB5)

    @pl.loop(0, nbat)
    def _(ob):
        crow = start_chunk + ob * KB5

        @pl.when(ob > 0)
        def _():
            pltpu.make_async_copy(wbuf, psp.at[irb.at[0]], ssem).wait()

        pltpu.sync_copy(is2d.at[pl.ds(crow, KB5)], isb)
        pltpu.sync_copy(ir2d.at[pl.ds(crow, KB5)], irb)
        pltpu.sync_copy(ex2d.at[pl.ds(crow, KB5)], exb)
        for j in range(KB5):
            if j > 0:
                pltpu.make_async_copy(wbuf, psp.at[irb.at[j]], ssem).wait()
            pltpu.async_copy(wne_hbm.at[isb.at[j]], wbuf, sem1).wait()

            @pl.loop(0, CH)
            def _(e):
                vec = exb[j, pl.ds(e * 16, 16)]
                for h in range(8):
                    bc = _lane_bcast(vec, h)
                    sl = pl.ds(h * 16, 16)
                    wbuf[e, sl] = wbuf[e, sl] * bc
                sl = pl.ds(D, 16)
                wbuf[e, sl] = wbuf[e, sl] * vec

            pltpu.async_copy(wbuf, psp.at[irb.at[j]], ssem, add=True)

    pltpu.make_async_copy(wbuf, psp.at[irb.at[0]], ssem).wait()
    plsc.subcore_barrier()
    for j in range(5):
        pltpu.sync_copy(psp.at[pl.ds(tb + j * 125, 125)],
                        pool_out.at[cid, pl.ds(tb + j * 125, 125)])


def _k5_call(wne, ex2d, is2d, ir2d):
    mesh = plsc.VectorSubcoreMesh(core_axis_name="c", subcore_axis_name="s")
    f = pl.kernel(
        _k5_body,
        out_type=jax.ShapeDtypeStruct((2, N, TW), _f32),
        mesh=mesh,
        compiler_params=pltpu.CompilerParams(use_tc_tiling_on_sc=False),
        scratch_types=[
            pltpu.VMEM((KB5, CH), jnp.int32),
            pltpu.VMEM((KB5, CH), jnp.int32),
            pltpu.VMEM((CH, TW), _f32),
            pltpu.VMEM((KB5, CH * 16), _f32),
            pltpu.VMEM_SHARED((N, TW), _f32),
            pltpu.SemaphoreType.DMA,
            pltpu.SemaphoreType.DMA,
        ],
    )
    return f(wne, ex2d, is2d, ir2d)


def _k6_body(pp_ref, nur_ref, nst_ref, gn, bn2, oh16, gruw, gruu, grub,
             o_ref):
    p = pp_ref[0] + pp_ref[1]
    praw = p[:, :D]
    dn16 = p[:, D:TW]
    dexp = lax.dot(dn16, oh16[...], preferred_element_type=_f32) + 1e-9
    att = jnp.maximum(praw / dexp, 0.0)
    s = nst_ref[...]
    mun = s[0:1, :] * (1.0 / N)
    msn = s[1:2, :] * (1.0 / N)
    varn = msn - mun * mun
    cn = gn[...] * lax.rsqrt(varn + EPS)
    dnn = bn2[...] - cn * mun
    nu = cn * nur_ref[...] + dnn
    gx = lax.dot(att, gruw[...], preferred_element_type=_f32) + grub[...]
    gh = lax.dot(nu, gruu[...], preferred_element_type=_f32)
    z = jax.nn.sigmoid(gx[:, :D] + gh[:, :D])
    r = jax.nn.sigmoid(gx[:, D:2 * D] + gh[:, D:2 * D])
    ht = jnp.tanh(gx[:, 2 * D:] + r * gh[:, 2 * D:])
    o_ref[...] = z * nu + (1.0 - z) * ht


def _k6_call(pool_part, nur, nstats, gn, bn2, oh16, gruw, gruu, grub):
    grid = (N // NB,)
    const = lambda shape: pl.BlockSpec(shape, lambda i: tuple(0 for _ in shape))
    return pl.pallas_call(
        _k6_body,
        grid=grid,
        in_specs=[
            pl.BlockSpec((2, NB, TW), lambda i: (0, i, 0)),
            pl.BlockSpec((NB, D), lambda i: (i, 0)),
            const((2, D)), const((1, D)), const((1, D)),
            const((16, D)), const((D, 3 * D)), const((D, 3 * D)),
            const((1, 3 * D)),
        ],
        out_specs=pl.BlockSpec((NB, D), lambda i: (i, 0)),
        out_shape=jax.ShapeDtypeStruct((N, D), _f32),
    )(pool_part, nur, nstats, gn, bn2, oh16, gruw, gruu, grub)


def kernel(node_attributes, edge_attributes, edge_indices, W_edge, b_edge,
           gamma_edge, beta_edge, W_att, b_att, a_att, W_node, b_node,
           gamma_node, beta_node, gru_W, gru_U, gru_b):
    idx_recv = edge_indices[:, 0]
    idx_send = edge_indices[:, 1]
    pad = E_PAD - E
    zpad = jnp.zeros((pad,), jnp.int32)
    is_p = jnp.concatenate([idx_send, zpad])
    ir_p = jnp.concatenate([idx_recv, zpad])

    w1 = W_edge[:D]
    w2 = W_edge[D:2 * D]
    w3 = W_edge[2 * D:]
    wcat = W_att.transpose(1, 0, 2).reshape(D, D)
    batf = b_att.reshape(1, D)

    blkid = jnp.arange(D) // DH
    col = jnp.arange(16)
    onehot = (blkid[:, None] == col[None, :]).astype(_f32)
    v_r = a_att[:, :DH].reshape(-1)
    v_s = a_att[:, DH:2 * DH].reshape(-1)
    ar16 = v_r[:, None] * onehot
    as16 = v_s[:, None] * onehot
    oh16 = onehot.T
    aet16 = jnp.concatenate(
        [a_att[:, 2 * DH:].T, jnp.zeros((D, 8), _f32)], axis=1)
    mask16 = (col < H).astype(_f32).reshape(1, 16)

    bedge = b_edge.reshape(1, D)
    gam = gamma_edge.reshape(1, D)
    bet = beta_edge.reshape(1, D)
    gn = gamma_node.reshape(1, D)
    bn2 = beta_node.reshape(1, D)
    bnod = b_node.reshape(1, D)
    grub = gru_b.reshape(1, 3 * D)

    ts, tr, wne, nur, nstats = _k1_call(
        node_attributes, w1, w2, wcat, batf, ar16, as16, W_node, bnod)
    is2d = is_p.reshape(E_PAD // CH, CH)
    ir2d = ir_p.reshape(E_PAD // CH, CH)
    gcat = _k2_call(ts, tr, is2d, ir2d)
    estats = _k3_call(gcat, edge_attributes, w3, bedge)
    y, expad = _k4_call(gcat, edge_attributes, estats, w3, bedge, gam, bet,
                        aet16, mask16)
    ex2d = jnp.concatenate(
        [expad, jnp.zeros((pad, 16), _f32)],
        axis=0).reshape(E_PAD // CH, CH * 16)
    pool_part = _k5_call(wne, ex2d, is2d, ir2d)
    node_final = _k6_call(pool_part, nur, nstats, gn, bn2, oh16,
                          gru_W, gru_U, grub)
    return node_final, y


# speedup vs baseline: 1.1654x; 1.0243x over previous
import jax
import jax.numpy as jnp
from jax import lax
from jax.experimental import pallas as pl
from jax.experimental.pallas import tpu as pltpu
from jax.experimental.pallas import tpu_sc as plsc

N = 10000
E = 320000
D = 128
DE = 16
H = 8
DH = 16
EPS = 1e-3

NW = 32
CH = 128
CPW = 80
E_PAD = NW * CH * CPW
TW = 144
TWB = 160
NPT = N // 16

EB = 6400
NB = 2000

_f32 = jnp.float32
_bf16 = jnp.bfloat16


def _k1_body(na_ref, w1, w2, wcat, batf, ar16, as16, wnod, bnod,
             ts_o, tr_o, wne_o, nur_o, nst_o, acc):
    i = pl.program_id(0)
    na = na_ref[...]
    ps = lax.dot(na, w1[...], preferred_element_type=_f32)
    pr = lax.dot(na, w2[...], preferred_element_type=_f32)
    wn = lax.dot(na, wcat[...], preferred_element_type=_f32) + batf[...]
    ls16 = lax.dot(wn, as16[...], preferred_element_type=_f32)
    lr16 = lax.dot(wn, ar16[...], preferred_element_type=_f32)
    blk = na.shape[0]
    zer16 = jnp.zeros((blk, 16), _f32)
    ts_o[...] = jnp.concatenate([ps, ls16, zer16], axis=1).astype(_bf16)
    tr_o[...] = jnp.concatenate([pr, lr16, zer16], axis=1).astype(_bf16)
    ones8 = jnp.ones((blk, 8), _f32)
    zer8 = jnp.zeros((blk, 8), _f32)
    wne_o[...] = jnp.concatenate([wn, ones8, zer8], axis=1)
    nu = jnp.maximum(lax.dot(na, wnod[...], preferred_element_type=_f32)
                     + bnod[...], 0.0)
    nur_o[...] = nu
    s0 = jnp.sum(nu, axis=0, keepdims=True)
    s1 = jnp.sum(nu * nu, axis=0, keepdims=True)
    st = jnp.concatenate([s0, s1], axis=0)

    @pl.when(i == 0)
    def _():
        acc[...] = jnp.zeros_like(acc)

    acc[...] += st

    @pl.when(i == pl.num_programs(0) - 1)
    def _():
        nst_o[...] = acc[...]


def _k1_call(na, w1, w2, wcat, batf, ar16, as16, wnod, bnod):
    nblk = 2000
    grid = (N // nblk,)
    const = lambda shape: pl.BlockSpec(shape, lambda i: (0, 0))
    return pl.pallas_call(
        _k1_body,
        grid=grid,
        in_specs=[
            pl.BlockSpec((nblk, D), lambda i: (i, 0)),
            const((D, D)), const((D, D)), const((D, D)), const((1, D)),
            const((D, 16)), const((D, 16)), const((D, D)), const((1, D)),
        ],
        out_specs=[
            pl.BlockSpec((nblk, TWB), lambda i: (i, 0)),
            pl.BlockSpec((nblk, TWB), lambda i: (i, 0)),
            pl.BlockSpec((nblk, TW), lambda i: (i, 0)),
            pl.BlockSpec((nblk, D), lambda i: (i, 0)),
            pl.BlockSpec((2, D), lambda i: (0, 0)),
        ],
        out_shape=[
            jax.ShapeDtypeStruct((N, TWB), _bf16),
            jax.ShapeDtypeStruct((N, TWB), _bf16),
            jax.ShapeDtypeStruct((N, TW), _f32),
            jax.ShapeDtypeStruct((N, D), _f32),
            jax.ShapeDtypeStruct((2, D), _f32),
        ],
        scratch_shapes=[pltpu.VMEM((2, D), _f32)],
    )(na, w1, w2, wcat, batf, ar16, as16, wnod, bnod)


KB2 = 8


def _k2_adds(bs, br):
    @pl.loop(0, CH)
    def _(e):
        for k in range(5):
            sl = pl.ds(k * 32, 32)
            bs[e, sl] = bs[e, sl] + br[e, sl]


def _k2_body(ts_hbm, tr_hbm, is2d, ir2d, gcat_hbm,
             isb, irb, bs, br, ts_sp, tr_sp, sg1, sg2):
    cid = lax.axis_index("c")
    sid = lax.axis_index("s")
    wid = sid * 2 + cid

    tb = sid * NPT
    pltpu.sync_copy(ts_hbm.at[pl.ds(tb, NPT)], ts_sp.at[pl.ds(tb, NPT)])
    pltpu.sync_copy(tr_hbm.at[pl.ds(tb, NPT)], tr_sp.at[pl.ds(tb, NPT)])
    plsc.subcore_barrier()

    @pl.loop(0, CPW // KB2)
    def _(ob):
        crow = wid * CPW + ob * KB2
        pltpu.sync_copy(is2d.at[pl.ds(crow, KB2)], isb)
        pltpu.sync_copy(ir2d.at[pl.ds(crow, KB2)], irb)
        for j in range(KB2):
            base = (crow + j) * CH
            c1 = pltpu.async_copy(ts_sp.at[isb.at[j]], bs, sg1)
            c2 = pltpu.async_copy(tr_sp.at[irb.at[j]], br, sg2)
            c1.wait()
            c2.wait()
            _k2_adds(bs, br)
            pltpu.sync_copy(bs, gcat_hbm.at[pl.ds(base, CH)])


def _k2_call(ts, tr, is2d, ir2d):
    mesh = plsc.VectorSubcoreMesh(core_axis_name="c", subcore_axis_name="s")
    f = pl.kernel(
        _k2_body,
        out_type=jax.ShapeDtypeStruct((E_PAD, TWB), _bf16),
        mesh=mesh,
        compiler_params=pltpu.CompilerParams(use_tc_tiling_on_sc=False),
        scratch_types=[
            pltpu.VMEM((KB2, CH), jnp.int32),
            pltpu.VMEM((KB2, CH), jnp.int32),
            pltpu.VMEM((CH, TWB), _bf16),
            pltpu.VMEM((CH, TWB), _bf16),
            pltpu.VMEM_SHARED((N, TWB), _bf16),
            pltpu.VMEM_SHARED((N, TWB), _bf16),
            pltpu.SemaphoreType.DMA,
            pltpu.SemaphoreType.DMA,
        ],
    )
    return f(ts, tr, is2d, ir2d)


def _k3_body(g_ref, ea_ref, w3, bedge, o_ref, acc):
    i = pl.program_id(0)
    g = g_ref[...][:, :D].astype(_f32)
    q = lax.dot(ea_ref[...], w3[...], preferred_element_type=_f32)
    x = jnp.maximum(g + q + bedge[...], 0.0)
    s0 = jnp.sum(x, axis=0, keepdims=True)
    s1 = jnp.sum(x * x, axis=0, keepdims=True)
    st = jnp.concatenate([s0, s1], axis=0)

    @pl.when(i == 0)
    def _():
        acc[...] = jnp.zeros_like(acc)

    acc[...] += st

    @pl.when(i == pl.num_programs(0) - 1)
    def _():
        o_ref[...] = acc[...]


def _k3_call(gcat, ea, w3, bedge):
    grid = (E // EB,)
    const = lambda shape: pl.BlockSpec(shape, lambda i: (0, 0))
    return pl.pallas_call(
        _k3_body,
        grid=grid,
        in_specs=[
            pl.BlockSpec((EB, TWB), lambda i: (i, 0)),
            pl.BlockSpec((EB, DE), lambda i: (i, 0)),
            const((DE, D)), const((1, D)),
        ],
        out_specs=pl.BlockSpec((2, D), lambda i: (0, 0)),
        out_shape=jax.ShapeDtypeStruct((2, D), _f32),
        scratch_shapes=[pltpu.VMEM((2, D), _f32)],
    )(gcat, ea, w3, bedge)


def _k4_body(g_ref, ea_ref, st_ref, w3, bedge, gam, bet, aet16, mask16,
             y_o, ex_o):
    s = st_ref[...]
    mu = s[0:1, :] * (1.0 / E)
    ms = s[1:2, :] * (1.0 / E)
    var = ms - mu * mu
    c = gam[...] * lax.rsqrt(var + EPS)
    d = bet[...] - c * mu
    gfull = g_ref[...]
    g = gfull[:, :D].astype(_f32)
    lsum16 = gfull[:, D:D + 16].astype(_f32)
    q = lax.dot(ea_ref[...], w3[...], preferred_element_type=_f32)
    x = jnp.maximum(g + q + bedge[...], 0.0)
    y = c * x + d
    y_o[...] = y
    u16 = lax.dot(y, aet16[...], preferred_element_type=_f32)
    v16 = lsum16 + u16
    lg16 = jnp.where(v16 >= 0.0, v16, 0.2 * v16)
    ex_o[...] = jnp.exp(lg16) * mask16[...]


def _k4_call(gcat, ea, estats, w3, bedge, gam, bet, aet16, mask16):
    grid = (E // EB,)
    const = lambda shape: pl.BlockSpec(shape, lambda i: (0, 0))
    return pl.pallas_call(
        _k4_body,
        grid=grid,
        in_specs=[
            pl.BlockSpec((EB, TWB), lambda i: (i, 0)),
            pl.BlockSpec((EB, DE), lambda i: (i, 0)),
            const((2, D)), const((DE, D)), const((1, D)),
            const((1, D)), const((1, D)), const((D, 16)), const((1, 16)),
        ],
        out_specs=[
            pl.BlockSpec((EB, D), lambda i: (i, 0)),
            pl.BlockSpec((EB, 16), lambda i: (i, 0)),
        ],
        out_shape=[
            jax.ShapeDtypeStruct((E, D), _f32),
            jax.ShapeDtypeStruct((E, 16), _f32),
        ],
    )(gcat, ea, estats, w3, bedge, gam, bet, aet16, mask16)


def _lane_bcast(vec, lane):
    dn = lax.GatherDimensionNumbers(
        offset_dims=(), collapsed_slice_dims=(0,), start_index_map=(0,))
    idx = jnp.full((16, 1), lane, jnp.int32)
    return lax.gather(vec, idx, dn, slice_sizes=(1,),
                      mode=lax.GatherScatterMode.PROMISE_IN_BOUNDS)


KB5 = 4
CPA = 100
CPB = 60


def _k5_body(wne_hbm, ex2d, is2d, ir2d, pool_out,
             isb, irb, wbuf, exb, psp, sem1, ssem):
    cid = lax.axis_index("c")
    sid = lax.axis_index("s")
    wid = sid * 2 + cid
    z16 = jnp.zeros((16,), _f32)

    @pl.loop(0, CH)
    def _(e):
        for k in range(9):
            wbuf[e, pl.ds(k * 16, 16)] = z16

    tb = sid * NPT
    for j in range(5):
        pltpu.sync_copy(wbuf.at[pl.ds(0, 125)],
                        psp.at[pl.ds(tb + j * 125, 125)])
    plsc.subcore_barrier()

    start_chunk = jnp.where(cid == 0, sid * CPA, 16 * CPA + sid * CPB)
    nbat = jnp.where(cid == 0, CPA // KB5, CPB // KB5)

    @pl.loop(0, nbat)
    def _(ob):
        crow = start_chunk + ob * KB5

        @pl.when(ob > 0)
        def _():
            pltpu.make_async_copy(wbuf, psp.at[irb.at[0]], ssem).wait()

        pltpu.sync_copy(is2d.at[pl.ds(crow, KB5)], isb)
        pltpu.sync_copy(ir2d.at[pl.ds(crow, KB5)], irb)
        pltpu.sync_copy(ex2d.at[pl.ds(crow, KB5)], exb)
        for j in range(KB5):
            if j > 0:
                pltpu.make_async_copy(wbuf, psp.at[irb.at[j]], ssem).wait()
            pltpu.async_copy(wne_hbm.at[isb.at[j]], wbuf, sem1).wait()

            @pl.loop(0, CH)
            def _(e):
                vec = exb[j, pl.ds(e * 16, 16)]
                for h in range(8):
                    bc = _lane_bcast(vec, h)
                    sl = pl.ds(h * 16, 16)
                    wbuf[e, sl] = wbuf[e, sl] * bc
                sl = pl.ds(D, 16)
                wbuf[e, sl] = wbuf[e, sl] * vec

            pltpu.async_copy(wbuf, psp.at[irb.at[j]], ssem, add=True)

    pltpu.make_async_copy(wbuf, psp.at[irb.at[0]], ssem).wait()
    plsc.subcore_barrier()
    for j in range(5):
        pltpu.sync_copy(psp.at[pl.ds(tb + j * 125, 125)],
                        pool_out.at[cid, pl.ds(tb + j * 125, 125)])


def _k5_call(wne, ex2d, is2d, ir2d):
    mesh = plsc.VectorSubcoreMesh(core_axis_name="c", subcore_axis_name="s")
    f = pl.kernel(
        _k5_body,
        out_type=jax.ShapeDtypeStruct((2, N, TW), _f32),
        mesh=mesh,
        compiler_params=pltpu.CompilerParams(use_tc_tiling_on_sc=False),
        scratch_types=[
            pltpu.VMEM((KB5, CH), jnp.int32),
            pltpu.VMEM((KB5, CH), jnp.int32),
            pltpu.VMEM((CH, TW), _f32),
            pltpu.VMEM((KB5, CH * 16), _f32),
            pltpu.VMEM_SHARED((N, TW), _f32),
            pltpu.SemaphoreType.DMA,
            pltpu.SemaphoreType.DMA,
        ],
    )
    return f(wne, ex2d, is2d, ir2d)


def _k6_body(pp_ref, nur_ref, nst_ref, gn, bn2, oh16, gruw, gruu, grub,
             o_ref):
    p = pp_ref[0] + pp_ref[1]
    praw = p[:, :D]
    dn16 = p[:, D:TW]
    dexp = lax.dot(dn16, oh16[...], preferred_element_type=_f32) + 1e-9
    att = jnp.maximum(praw / dexp, 0.0)
    s = nst_ref[...]
    mun = s[0:1, :] * (1.0 / N)
    msn = s[1:2, :] * (1.0 / N)
    varn = msn - mun * mun
    cn = gn[...] * lax.rsqrt(varn + EPS)
    dnn = bn2[...] - cn * mun
    nu = cn * nur_ref[...] + dnn
    gx = lax.dot(att, gruw[...], preferred_element_type=_f32) + grub[...]
    gh = lax.dot(nu, gruu[...], preferred_element_type=_f32)
    z = jax.nn.sigmoid(gx[:, :D] + gh[:, :D])
    r = jax.nn.sigmoid(gx[:, D:2 * D] + gh[:, D:2 * D])
    ht = jnp.tanh(gx[:, 2 * D:] + r * gh[:, 2 * D:])
    o_ref[...] = z * nu + (1.0 - z) * ht


def _k6_call(pool_part, nur, nstats, gn, bn2, oh16, gruw, gruu, grub):
    grid = (N // NB,)
    const = lambda shape: pl.BlockSpec(shape, lambda i: tuple(0 for _ in shape))
    return pl.pallas_call(
        _k6_body,
        grid=grid,
        in_specs=[
            pl.BlockSpec((2, NB, TW), lambda i: (0, i, 0)),
            pl.BlockSpec((NB, D), lambda i: (i, 0)),
            const((2, D)), const((1, D)), const((1, D)),
            const((16, D)), const((D, 3 * D)), const((D, 3 * D)),
            const((1, 3 * D)),
        ],
        out_specs=pl.BlockSpec((NB, D), lambda i: (i, 0)),
        out_shape=jax.ShapeDtypeStruct((N, D), _f32),
    )(pool_part, nur, nstats, gn, bn2, oh16, gruw, gruu, grub)


def kernel(node_attributes, edge_attributes, edge_indices, W_edge, b_edge,
           gamma_edge, beta_edge, W_att, b_att, a_att, W_node, b_node,
           gamma_node, beta_node, gru_W, gru_U, gru_b):
    idx_recv = edge_indices[:, 0]
    idx_send = edge_indices[:, 1]
    pad = E_PAD - E
    zpad = jnp.zeros((pad,), jnp.int32)
    is_p = jnp.concatenate([idx_send, zpad])
    ir_p = jnp.concatenate([idx_recv, zpad])

    w1 = W_edge[:D]
    w2 = W_edge[D:2 * D]
    w3 = W_edge[2 * D:]
    wcat = W_att.transpose(1, 0, 2).reshape(D, D)
    batf = b_att.reshape(1, D)

    blkid = jnp.arange(D) // DH
    col = jnp.arange(16)
    onehot = (blkid[:, None] == col[None, :]).astype(_f32)
    v_r = a_att[:, :DH].reshape(-1)
    v_s = a_att[:, DH:2 * DH].reshape(-1)
    ar16 = v_r[:, None] * onehot
    as16 = v_s[:, None] * onehot
    oh16 = onehot.T
    aet16 = jnp.concatenate(
        [a_att[:, 2 * DH:].T, jnp.zeros((D, 8), _f32)], axis=1)
    mask16 = (col < H).astype(_f32).reshape(1, 16)

    bedge = b_edge.reshape(1, D)
    gam = gamma_edge.reshape(1, D)
    bet = beta_edge.reshape(1, D)
    gn = gamma_node.reshape(1, D)
    bn2 = beta_node.reshape(1, D)
    bnod = b_node.reshape(1, D)
    grub = gru_b.reshape(1, 3 * D)

    ts, tr, wne, nur, nstats = _k1_call(
        node_attributes, w1, w2, wcat, batf, ar16, as16, W_node, bnod)
    is2d = is_p.reshape(E_PAD // CH, CH)
    ir2d = ir_p.reshape(E_PAD // CH, CH)
    gcat = _k2_call(ts, tr, is2d, ir2d)
    estats = _k3_call(gcat, edge_attributes, w3, bedge)
    y, expad = _k4_call(gcat, edge_attributes, estats, w3, bedge, gam, bet,
                        aet16, mask16)
    ex2d = jnp.concatenate(
        [expad, jnp.zeros((pad, 16), _f32)],
        axis=0).reshape(E_PAD // CH, CH * 16)
    pool_part = _k5_call(wne, ex2d, is2d, ir2d)
    node_final = _k6_call(pool_part, nur, nstats, gn, bn2, oh16,
                          gru_W, gru_U, grub)
    return node_final, y


# speedup vs baseline: 1.1672x; 1.0016x over previous
import jax
import jax.numpy as jnp
from jax import lax
from jax.experimental import pallas as pl
from jax.experimental.pallas import tpu as pltpu
from jax.experimental.pallas import tpu_sc as plsc

N = 10000
E = 320000
D = 128
DE = 16
H = 8
DH = 16
EPS = 1e-3

NW = 32
CH = 128
CPW = 80
E_PAD = NW * CH * CPW
TW = 144
TWB = 160
NPT = N // 16

EB = 12800
NB = 2000

_f32 = jnp.float32
_bf16 = jnp.bfloat16


def _k1_body(na_ref, w1, w2, wcat, batf, ar16, as16, wnod, bnod,
             ts_o, tr_o, wne_o, nur_o, nst_o, acc):
    i = pl.program_id(0)
    na = na_ref[...]
    ps = lax.dot(na, w1[...], preferred_element_type=_f32)
    pr = lax.dot(na, w2[...], preferred_element_type=_f32)
    wn = lax.dot(na, wcat[...], preferred_element_type=_f32) + batf[...]
    ls16 = lax.dot(wn, as16[...], preferred_element_type=_f32)
    lr16 = lax.dot(wn, ar16[...], preferred_element_type=_f32)
    blk = na.shape[0]
    zer16 = jnp.zeros((blk, 16), _f32)
    ts_o[...] = jnp.concatenate([ps, ls16, zer16], axis=1).astype(_bf16)
    tr_o[...] = jnp.concatenate([pr, lr16, zer16], axis=1).astype(_bf16)
    ones8 = jnp.ones((blk, 8), _f32)
    zer8 = jnp.zeros((blk, 8), _f32)
    wne_o[...] = jnp.concatenate([wn, ones8, zer8], axis=1)
    nu = jnp.maximum(lax.dot(na, wnod[...], preferred_element_type=_f32)
                     + bnod[...], 0.0)
    nur_o[...] = nu
    s0 = jnp.sum(nu, axis=0, keepdims=True)
    s1 = jnp.sum(nu * nu, axis=0, keepdims=True)
    st = jnp.concatenate([s0, s1], axis=0)

    @pl.when(i == 0)
    def _():
        acc[...] = jnp.zeros_like(acc)

    acc[...] += st

    @pl.when(i == pl.num_programs(0) - 1)
    def _():
        nst_o[...] = acc[...]


def _k1_call(na, w1, w2, wcat, batf, ar16, as16, wnod, bnod):
    nblk = 2000
    grid = (N // nblk,)
    const = lambda shape: pl.BlockSpec(shape, lambda i: (0, 0))
    return pl.pallas_call(
        _k1_body,
        grid=grid,
        in_specs=[
            pl.BlockSpec((nblk, D), lambda i: (i, 0)),
            const((D, D)), const((D, D)), const((D, D)), const((1, D)),
            const((D, 16)), const((D, 16)), const((D, D)), const((1, D)),
        ],
        out_specs=[
            pl.BlockSpec((nblk, TWB), lambda i: (i, 0)),
            pl.BlockSpec((nblk, TWB), lambda i: (i, 0)),
            pl.BlockSpec((nblk, TW), lambda i: (i, 0)),
            pl.BlockSpec((nblk, D), lambda i: (i, 0)),
            pl.BlockSpec((2, D), lambda i: (0, 0)),
        ],
        out_shape=[
            jax.ShapeDtypeStruct((N, TWB), _bf16),
            jax.ShapeDtypeStruct((N, TWB), _bf16),
            jax.ShapeDtypeStruct((N, TW), _f32),
            jax.ShapeDtypeStruct((N, D), _f32),
            jax.ShapeDtypeStruct((2, D), _f32),
        ],
        scratch_shapes=[pltpu.VMEM((2, D), _f32)],
    )(na, w1, w2, wcat, batf, ar16, as16, wnod, bnod)


KB2 = 8


def _k2_adds(bs, br):
    @pl.loop(0, CH)
    def _(e):
        for k in range(5):
            sl = pl.ds(k * 32, 32)
            bs[e, sl] = bs[e, sl] + br[e, sl]


def _k2_body(ts_hbm, tr_hbm, is2d, ir2d, gcat_hbm,
             isb, irb, bs, br, ts_sp, tr_sp, sg1, sg2):
    cid = lax.axis_index("c")
    sid = lax.axis_index("s")
    wid = sid * 2 + cid

    tb = sid * NPT
    pltpu.sync_copy(ts_hbm.at[pl.ds(tb, NPT)], ts_sp.at[pl.ds(tb, NPT)])
    pltpu.sync_copy(tr_hbm.at[pl.ds(tb, NPT)], tr_sp.at[pl.ds(tb, NPT)])
    plsc.subcore_barrier()

    @pl.loop(0, CPW // KB2)
    def _(ob):
        crow = wid * CPW + ob * KB2
        pltpu.sync_copy(is2d.at[pl.ds(crow, KB2)], isb)
        pltpu.sync_copy(ir2d.at[pl.ds(crow, KB2)], irb)
        for j in range(KB2):
            base = (crow + j) * CH
            c1 = pltpu.async_copy(ts_sp.at[isb.at[j]], bs, sg1)
            c2 = pltpu.async_copy(tr_sp.at[irb.at[j]], br, sg2)
            c1.wait()
            c2.wait()
            _k2_adds(bs, br)
            pltpu.sync_copy(bs, gcat_hbm.at[pl.ds(base, CH)])


def _k2_call(ts, tr, is2d, ir2d):
    mesh = plsc.VectorSubcoreMesh(core_axis_name="c", subcore_axis_name="s")
    f = pl.kernel(
        _k2_body,
        out_type=jax.ShapeDtypeStruct((E_PAD, TWB), _bf16),
        mesh=mesh,
        compiler_params=pltpu.CompilerParams(use_tc_tiling_on_sc=False),
        scratch_types=[
            pltpu.VMEM((KB2, CH), jnp.int32),
            pltpu.VMEM((KB2, CH), jnp.int32),
            pltpu.VMEM((CH, TWB), _bf16),
            pltpu.VMEM((CH, TWB), _bf16),
            pltpu.VMEM_SHARED((N, TWB), _bf16),
            pltpu.VMEM_SHARED((N, TWB), _bf16),
            pltpu.SemaphoreType.DMA,
            pltpu.SemaphoreType.DMA,
        ],
    )
    return f(ts, tr, is2d, ir2d)


def _k3_body(g_ref, ea_ref, w3, bedge, o_ref, acc):
    i = pl.program_id(0)
    g = g_ref[...][:, :D].astype(_f32)
    q = lax.dot(ea_ref[...], w3[...], preferred_element_type=_f32)
    x = jnp.maximum(g + q + bedge[...], 0.0)
    s0 = jnp.sum(x, axis=0, keepdims=True)
    s1 = jnp.sum(x * x, axis=0, keepdims=True)
    st = jnp.concatenate([s0, s1], axis=0)

    @pl.when(i == 0)
    def _():
        acc[...] = jnp.zeros_like(acc)

    acc[...] += st

    @pl.when(i == pl.num_programs(0) - 1)
    def _():
        o_ref[...] = acc[...]


def _k3_call(gcat, ea, w3, bedge):
    grid = (E // EB,)
    const = lambda shape: pl.BlockSpec(shape, lambda i: (0, 0))
    return pl.pallas_call(
        _k3_body,
        grid=grid,
        in_specs=[
            pl.BlockSpec((EB, TWB), lambda i: (i, 0)),
            pl.BlockSpec((EB, DE), lambda i: (i, 0)),
            const((DE, D)), const((1, D)),
        ],
        out_specs=pl.BlockSpec((2, D), lambda i: (0, 0)),
        out_shape=jax.ShapeDtypeStruct((2, D), _f32),
        scratch_shapes=[pltpu.VMEM((2, D), _f32)],
    )(gcat, ea, w3, bedge)


def _k4_body(g_ref, ea_ref, st_ref, w3, bedge, gam, bet, aet16, mask16,
             y_o, ex_o):
    s = st_ref[...]
    mu = s[0:1, :] * (1.0 / E)
    ms = s[1:2, :] * (1.0 / E)
    var = ms - mu * mu
    c = gam[...] * lax.rsqrt(var + EPS)
    d = bet[...] - c * mu
    gfull = g_ref[...]
    g = gfull[:, :D].astype(_f32)
    lsum16 = gfull[:, D:D + 16].astype(_f32)
    q = lax.dot(ea_ref[...], w3[...], preferred_element_type=_f32)
    x = jnp.maximum(g + q + bedge[...], 0.0)
    y = c * x + d
    y_o[...] = y
    u16 = lax.dot(y, aet16[...], preferred_element_type=_f32)
    v16 = lsum16 + u16
    lg16 = jnp.where(v16 >= 0.0, v16, 0.2 * v16)
    ex_o[...] = jnp.exp(lg16) * mask16[...]


def _k4_call(gcat, ea, estats, w3, bedge, gam, bet, aet16, mask16):
    grid = (E // EB,)
    const = lambda shape: pl.BlockSpec(shape, lambda i: (0, 0))
    return pl.pallas_call(
        _k4_body,
        grid=grid,
        in_specs=[
            pl.BlockSpec((EB, TWB), lambda i: (i, 0)),
            pl.BlockSpec((EB, DE), lambda i: (i, 0)),
            const((2, D)), const((DE, D)), const((1, D)),
            const((1, D)), const((1, D)), const((D, 16)), const((1, 16)),
        ],
        out_specs=[
            pl.BlockSpec((EB, D), lambda i: (i, 0)),
            pl.BlockSpec((EB, 16), lambda i: (i, 0)),
        ],
        out_shape=[
            jax.ShapeDtypeStruct((E, D), _f32),
            jax.ShapeDtypeStruct((E, 16), _f32),
        ],
    )(gcat, ea, estats, w3, bedge, gam, bet, aet16, mask16)


def _lane_bcast(vec, lane):
    dn = lax.GatherDimensionNumbers(
        offset_dims=(), collapsed_slice_dims=(0,), start_index_map=(0,))
    idx = jnp.full((16, 1), lane, jnp.int32)
    return lax.gather(vec, idx, dn, slice_sizes=(1,),
                      mode=lax.GatherScatterMode.PROMISE_IN_BOUNDS)


KB5 = 4
CPA = 100
CPB = 60


def _k5_body(wne_hbm, ex2d, is2d, ir2d, pool_out,
             isb, irb, wbuf, exb, psp, sem1, ssem):
    cid = lax.axis_index("c")
    sid = lax.axis_index("s")
    wid = sid * 2 + cid
    z16 = jnp.zeros((16,), _f32)

    @pl.loop(0, CH)
    def _(e):
        for k in range(9):
            wbuf[e, pl.ds(k * 16, 16)] = z16

    tb = sid * NPT
    for j in range(5):
        pltpu.sync_copy(wbuf.at[pl.ds(0, 125)],
                        psp.at[pl.ds(tb + j * 125, 125)])
    plsc.subcore_barrier()

    start_chunk = jnp.where(cid == 0, sid * CPA, 16 * CPA + sid * CPB)
    nbat = jnp.where(cid == 0, CPA // KB5, CPB // KB5)

    @pl.loop(0, nbat)
    def _(ob):
        crow = start_chunk + ob * KB5

        @pl.when(ob > 0)
        def _():
            pltpu.make_async_copy(wbuf, psp.at[irb.at[0]], ssem).wait()

        pltpu.sync_copy(is2d.at[pl.ds(crow, KB5)], isb)
        pltpu.sync_copy(ir2d.at[pl.ds(crow, KB5)], irb)
        pltpu.sync_copy(ex2d.at[pl.ds(crow, KB5)], exb)
        for j in range(KB5):
            if j > 0:
                pltpu.make_async_copy(wbuf, psp.at[irb.at[j]], ssem).wait()
            pltpu.async_copy(wne_hbm.at[isb.at[j]], wbuf, sem1).wait()

            @pl.loop(0, CH)
            def _(e):
                vec = exb[j, pl.ds(e * 16, 16)]
                for h in range(8):
                    bc = _lane_bcast(vec, h)
                    sl = pl.ds(h * 16, 16)
                    wbuf[e, sl] = wbuf[e, sl] * bc
                sl = pl.ds(D, 16)
                wbuf[e, sl] = wbuf[e, sl] * vec

            pltpu.async_copy(wbuf, psp.at[irb.at[j]], ssem, add=True)

    pltpu.make_async_copy(wbuf, psp.at[irb.at[0]], ssem).wait()
    plsc.subcore_barrier()
    for j in range(5):
        pltpu.sync_copy(psp.at[pl.ds(tb + j * 125, 125)],
                        pool_out.at[cid, pl.ds(tb + j * 125, 125)])


def _k5_call(wne, ex2d, is2d, ir2d):
    mesh = plsc.VectorSubcoreMesh(core_axis_name="c", subcore_axis_name="s")
    f = pl.kernel(
        _k5_body,
        out_type=jax.ShapeDtypeStruct((2, N, TW), _f32),
        mesh=mesh,
        compiler_params=pltpu.CompilerParams(use_tc_tiling_on_sc=False),
        scratch_types=[
            pltpu.VMEM((KB5, CH), jnp.int32),
            pltpu.VMEM((KB5, CH), jnp.int32),
            pltpu.VMEM((CH, TW), _f32),
            pltpu.VMEM((KB5, CH * 16), _f32),
            pltpu.VMEM_SHARED((N, TW), _f32),
            pltpu.SemaphoreType.DMA,
            pltpu.SemaphoreType.DMA,
        ],
    )
    return f(wne, ex2d, is2d, ir2d)


def _k6_body(pp_ref, nur_ref, nst_ref, gn, bn2, oh16, gruw, gruu, grub,
             o_ref):
    p = pp_ref[0] + pp_ref[1]
    praw = p[:, :D]
    dn16 = p[:, D:TW]
    dexp = lax.dot(dn16, oh16[...], preferred_element_type=_f32) + 1e-9
    att = jnp.maximum(praw / dexp, 0.0)
    s = nst_ref[...]
    mun = s[0:1, :] * (1.0 / N)
    msn = s[1:2, :] * (1.0 / N)
    varn = msn - mun * mun
    cn = gn[...] * lax.rsqrt(varn + EPS)
    dnn = bn2[...] - cn * mun
    nu = cn * nur_ref[...] + dnn
    gx = lax.dot(att, gruw[...], preferred_element_type=_f32) + grub[...]
    gh = lax.dot(nu, gruu[...], preferred_element_type=_f32)
    z = jax.nn.sigmoid(gx[:, :D] + gh[:, :D])
    r = jax.nn.sigmoid(gx[:, D:2 * D] + gh[:, D:2 * D])
    ht = jnp.tanh(gx[:, 2 * D:] + r * gh[:, 2 * D:])
    o_ref[...] = z * nu + (1.0 - z) * ht


def _k6_call(pool_part, nur, nstats, gn, bn2, oh16, gruw, gruu, grub):
    grid = (N // NB,)
    const = lambda shape: pl.BlockSpec(shape, lambda i: tuple(0 for _ in shape))
    return pl.pallas_call(
        _k6_body,
        grid=grid,
        in_specs=[
            pl.BlockSpec((2, NB, TW), lambda i: (0, i, 0)),
            pl.BlockSpec((NB, D), lambda i: (i, 0)),
            const((2, D)), const((1, D)), const((1, D)),
            const((16, D)), const((D, 3 * D)), const((D, 3 * D)),
            const((1, 3 * D)),
        ],
        out_specs=pl.BlockSpec((NB, D), lambda i: (i, 0)),
        out_shape=jax.ShapeDtypeStruct((N, D), _f32),
    )(pool_part, nur, nstats, gn, bn2, oh16, gruw, gruu, grub)


def kernel(node_attributes, edge_attributes, edge_indices, W_edge, b_edge,
           gamma_edge, beta_edge, W_att, b_att, a_att, W_node, b_node,
           gamma_node, beta_node, gru_W, gru_U, gru_b):
    idx_recv = edge_indices[:, 0]
    idx_send = edge_indices[:, 1]
    pad = E_PAD - E
    zpad = jnp.zeros((pad,), jnp.int32)
    is_p = jnp.concatenate([idx_send, zpad])
    ir_p = jnp.concatenate([idx_recv, zpad])

    w1 = W_edge[:D]
    w2 = W_edge[D:2 * D]
    w3 = W_edge[2 * D:]
    wcat = W_att.transpose(1, 0, 2).reshape(D, D)
    batf = b_att.reshape(1, D)

    blkid = jnp.arange(D) // DH
    col = jnp.arange(16)
    onehot = (blkid[:, None] == col[None, :]).astype(_f32)
    v_r = a_att[:, :DH].reshape(-1)
    v_s = a_att[:, DH:2 * DH].reshape(-1)
    ar16 = v_r[:, None] * onehot
    as16 = v_s[:, None] * onehot
    oh16 = onehot.T
    aet16 = jnp.concatenate(
        [a_att[:, 2 * DH:].T, jnp.zeros((D, 8), _f32)], axis=1)
    mask16 = (col < H).astype(_f32).reshape(1, 16)

    bedge = b_edge.reshape(1, D)
    gam = gamma_edge.reshape(1, D)
    bet = beta_edge.reshape(1, D)
    gn = gamma_node.reshape(1, D)
    bn2 = beta_node.reshape(1, D)
    bnod = b_node.reshape(1, D)
    grub = gru_b.reshape(1, 3 * D)

    ts, tr, wne, nur, nstats = _k1_call(
        node_attributes, w1, w2, wcat, batf, ar16, as16, W_node, bnod)
    is2d = is_p.reshape(E_PAD // CH, CH)
    ir2d = ir_p.reshape(E_PAD // CH, CH)
    gcat = _k2_call(ts, tr, is2d, ir2d)
    estats = _k3_call(gcat, edge_attributes, w3, bedge)
    y, expad = _k4_call(gcat, edge_attributes, estats, w3, bedge, gam, bet,
                        aet16, mask16)
    ex2d = jnp.concatenate(
        [expad, jnp.zeros((pad, 16), _f32)],
        axis=0).reshape(E_PAD // CH, CH * 16)
    pool_part = _k5_call(wne, ex2d, is2d, ir2d)
    node_final = _k6_call(pool_part, nur, nstats, gn, bn2, oh16,
                          gru_W, gru_U, grub)
    return node_final, y
